# initial kernel scaffold (unmeasured)
import jax
import jax.numpy as jnp
from jax import lax
from jax.experimental import pallas as pl
from jax.experimental.pallas import tpu as pltpu


def kernel(
    x,
):
    def body(*refs):
        pass

    out_shape = jax.ShapeDtypeStruct(..., jnp.float32)
    return pl.pallas_call(body, out_shape=out_shape)(...)


# baseline (device time: 16182 ns/iter reference)
import functools

import jax
import jax.numpy as jnp
from jax import lax
from jax.experimental import pallas as pl
from jax.experimental.pallas import tpu as pltpu

N_DEV = 16
M_GLOBAL = 24576


def kernel(x):
    m_per, n = x.shape

    def body(x_ref, out_ref, mine_ref, comm_ref, send_sems, recv_sems):
        my_pos = lax.axis_index("i")

        partial = jnp.sum(x_ref[...], axis=0, keepdims=True)
        mine_ref[...] = partial
        comm_ref[pl.ds(my_pos, 1), :] = partial

        barrier_sem = pltpu.get_barrier_semaphore()
        for d in range(1, N_DEV):
            tgt = lax.rem(my_pos + d, N_DEV)
            pl.semaphore_signal(
                barrier_sem, inc=1,
                device_id=(tgt,), device_id_type=pl.DeviceIdType.MESH,
            )
        pl.semaphore_wait(barrier_sem, N_DEV - 1)

        rdmas = []
        for d in range(1, N_DEV):
            tgt = lax.rem(my_pos + d, N_DEV)
            rdma = pltpu.make_async_remote_copy(
                src_ref=mine_ref,
                dst_ref=comm_ref.at[pl.ds(my_pos, 1), :],
                send_sem=send_sems.at[d - 1],
                recv_sem=recv_sems.at[d - 1],
                device_id=(tgt,),
                device_id_type=pl.DeviceIdType.MESH,
            )
            rdma.start()
            rdmas.append(rdma)

        for rdma in rdmas:
            rdma.wait_recv()
        for rdma in rdmas:
            rdma.wait_send()

        out_ref[...] = jnp.sum(comm_ref[...], axis=0, keepdims=True) * (
            1.0 / M_GLOBAL
        )

        @functools.partial(
            pl.run_scoped, exit_sem=pltpu.SemaphoreType.REGULAR
        )
        def _(exit_sem):
            for d in range(1, N_DEV):
                tgt = lax.rem(my_pos + d, N_DEV)
                pl.semaphore_signal(
                    exit_sem, inc=1,
                    device_id=(tgt,), device_id_type=pl.DeviceIdType.MESH,
                )
            pl.semaphore_wait(exit_sem, N_DEV - 1)

    return pl.pallas_call(
        body,
        out_shape=jax.ShapeDtypeStruct((1, n), jnp.float32),
        in_specs=[pl.BlockSpec(memory_space=pltpu.VMEM)],
        out_specs=pl.BlockSpec(memory_space=pltpu.VMEM),
        scratch_shapes=[
            pltpu.VMEM((1, n), jnp.float32),
            pltpu.VMEM((N_DEV, n), jnp.float32),
            pltpu.SemaphoreType.DMA((N_DEV - 1,)),
            pltpu.SemaphoreType.DMA((N_DEV - 1,)),
        ],
        compiler_params=pltpu.CompilerParams(collective_id=0),
    )(x)


# device time: 15924 ns/iter; 1.0162x vs baseline; 1.0162x over previous
import functools

import jax
import jax.numpy as jnp
from jax import lax
from jax.experimental import pallas as pl
from jax.experimental.pallas import tpu as pltpu

N_DEV = 16
M_GLOBAL = 24576
N_CHUNKS = 6


def kernel(x):
    m_per, n = x.shape
    chunk = m_per // N_CHUNKS
    assert chunk * N_CHUNKS == m_per

    def body(
        x_hbm,
        out_ref,
        chunk_buf,
        mine_ref,
        comm_ref,
        copy_sems,
        send_sems,
        recv_sems,
    ):
        my_pos = lax.axis_index("i")

        barrier_sem = pltpu.get_barrier_semaphore()
        for d in range(1, N_DEV):
            tgt = lax.rem(my_pos + d, N_DEV)
            pl.semaphore_signal(
                barrier_sem, inc=1,
                device_id=(tgt,), device_id_type=pl.DeviceIdType.MESH,
            )

        def copy_in(c):
            return pltpu.make_async_copy(
                x_hbm.at[pl.ds(c * chunk, chunk), :],
                chunk_buf.at[c % 2],
                copy_sems.at[c % 2],
            )

        copy_in(0).start()
        copy_in(1).start()
        partial = jnp.zeros((1, n), jnp.float32)
        for c in range(N_CHUNKS):
            copy_in(c).wait()
            partial = partial + jnp.sum(
                chunk_buf[c % 2], axis=0, keepdims=True
            )
            if c + 2 < N_CHUNKS:
                copy_in(c + 2).start()
        mine_ref[...] = partial
        comm_ref[pl.ds(my_pos, 1), :] = partial

        pl.semaphore_wait(barrier_sem, N_DEV - 1)

        rdmas = []
        for d in range(1, N_DEV):
            tgt = lax.rem(my_pos + d, N_DEV)
            rdma = pltpu.make_async_remote_copy(
                src_ref=mine_ref,
                dst_ref=comm_ref.at[pl.ds(my_pos, 1), :],
                send_sem=send_sems.at[d - 1],
                recv_sem=recv_sems.at[d - 1],
                device_id=(tgt,),
                device_id_type=pl.DeviceIdType.MESH,
            )
            rdma.start()
            rdmas.append(rdma)

        for rdma in rdmas:
            rdma.wait_recv()

        @functools.partial(
            pl.run_scoped, exit_sem=pltpu.SemaphoreType.REGULAR
        )
        def _(exit_sem):
            for d in range(1, N_DEV):
                tgt = lax.rem(my_pos + d, N_DEV)
                pl.semaphore_signal(
                    exit_sem, inc=1,
                    device_id=(tgt,), device_id_type=pl.DeviceIdType.MESH,
                )

            out_ref[...] = jnp.sum(
                comm_ref[...], axis=0, keepdims=True
            ) * (1.0 / M_GLOBAL)

            for rdma in rdmas:
                rdma.wait_send()
            pl.semaphore_wait(exit_sem, N_DEV - 1)

    return pl.pallas_call(
        body,
        out_shape=jax.ShapeDtypeStruct((1, n), jnp.float32),
        in_specs=[pl.BlockSpec(memory_space=pl.ANY)],
        out_specs=pl.BlockSpec(memory_space=pltpu.VMEM),
        scratch_shapes=[
            pltpu.VMEM((2, chunk, n), jnp.float32),
            pltpu.VMEM((1, n), jnp.float32),
            pltpu.VMEM((N_DEV, n), jnp.float32),
            pltpu.SemaphoreType.DMA((2,)),
            pltpu.SemaphoreType.DMA((N_DEV - 1,)),
            pltpu.SemaphoreType.DMA((N_DEV - 1,)),
        ],
        compiler_params=pltpu.CompilerParams(collective_id=0),
    )(x)


# device time: 11488 ns/iter; 1.4086x vs baseline; 1.3861x over previous
import jax
import jax.numpy as jnp
from jax import lax
from jax.experimental import pallas as pl
from jax.experimental.pallas import tpu as pltpu

N_DEV = 16
M_GLOBAL = 24576
N_CHUNKS = 6


def kernel(x):
    m_per, n = x.shape
    chunk = m_per // N_CHUNKS
    assert chunk * N_CHUNKS == m_per

    def body(
        x_hbm,
        out_ref,
        chunk_buf,
        mine_ref,
        comm_ref,
        copy_sems,
        send_sems,
        recv_sems,
        credit_sems,
    ):
        my_pos = lax.axis_index("i")

        barrier_sem = pltpu.get_barrier_semaphore()
        for d in range(1, N_DEV):
            tgt = lax.rem(my_pos + d, N_DEV)
            pl.semaphore_signal(
                barrier_sem, inc=1,
                device_id=(tgt,), device_id_type=pl.DeviceIdType.MESH,
            )
            pl.semaphore_signal(
                credit_sems.at[d - 1], inc=1,
                device_id=(tgt,), device_id_type=pl.DeviceIdType.MESH,
            )

        def copy_in(c):
            return pltpu.make_async_copy(
                x_hbm.at[pl.ds(c * chunk, chunk), :],
                chunk_buf.at[c % 2],
                copy_sems.at[c % 2],
            )

        copy_in(0).start()
        copy_in(1).start()
        partial = jnp.zeros((1, n), jnp.float32)
        for c in range(N_CHUNKS):
            copy_in(c).wait()
            partial = partial + jnp.sum(
                chunk_buf[c % 2], axis=0, keepdims=True
            )
            if c + 2 < N_CHUNKS:
                copy_in(c + 2).start()
        mine_ref[...] = partial
        comm_ref[pl.ds(my_pos, 1), :] = partial

        pl.semaphore_wait(barrier_sem, N_DEV - 1)

        rdmas = []
        for d in range(1, N_DEV):
            tgt = lax.rem(my_pos + d, N_DEV)
            pl.semaphore_wait(credit_sems.at[(N_DEV - d) - 1], 1)
            rdma = pltpu.make_async_remote_copy(
                src_ref=mine_ref,
                dst_ref=comm_ref.at[pl.ds(my_pos, 1), :],
                send_sem=send_sems.at[d - 1],
                recv_sem=recv_sems.at[d - 1],
                device_id=(tgt,),
                device_id_type=pl.DeviceIdType.MESH,
            )
            rdma.start()
            rdmas.append(rdma)

        for rdma in rdmas:
            rdma.wait_recv()

        out_ref[...] = jnp.sum(comm_ref[...], axis=0, keepdims=True) * (
            1.0 / M_GLOBAL
        )

        for rdma in rdmas:
            rdma.wait_send()

    return pl.pallas_call(
        body,
        out_shape=jax.ShapeDtypeStruct((1, n), jnp.float32),
        in_specs=[pl.BlockSpec(memory_space=pl.ANY)],
        out_specs=pl.BlockSpec(memory_space=pltpu.VMEM),
        scratch_shapes=[
            pltpu.VMEM((2, chunk, n), jnp.float32),
            pltpu.VMEM((1, n), jnp.float32),
            pltpu.VMEM((N_DEV, n), jnp.float32),
            pltpu.SemaphoreType.DMA((2,)),
            pltpu.SemaphoreType.DMA((N_DEV - 1,)),
            pltpu.SemaphoreType.DMA((N_DEV - 1,)),
            pltpu.SemaphoreType.REGULAR((N_DEV - 1,)),
        ],
        compiler_params=pltpu.CompilerParams(collective_id=0),
    )(x)
